# baseline (speedup 1.0000x reference)
import jax
import jax.numpy as jnp
from jax import lax
from jax.experimental import pallas as pl
from jax.experimental.pallas import tpu as pltpu
from jax.experimental.pallas import tpu_sc as plsc

B = 16384
D = 128
T = 100000
NC, NS, L = 2, 16, 16
NW = NC * NS
N_BIG = 20
R_BIG, R_SMALL = 3128, 3120
WPOS_PAD = 3136
COMP_PAD = 3200
C = 128
MOM = 0.1


def _body(feat_hbm, idx_hbm, lab_hbm, mem_hbm, cls_hbm,
          out_feat, out_cls,
          idx_v, lab_v, wpos, comp_pos, comp_loc, glob2d, cls_v,
          fbuf, obuf, rot, sem_cp, sem_g0, sem_g1, sem_sc):
    w = lax.axis_index("s") * NC + lax.axis_index("c")
    big = w < N_BIG
    base = jnp.where(big, w * R_BIG, N_BIG * R_BIG + (w - N_BIG) * R_SMALL)
    nrows = jnp.where(big, R_BIG, R_SMALL)
    iota = lax.iota(jnp.int32, L)

    @pl.when(big)
    def _():
        pltpu.async_copy(mem_hbm.at[pl.ds(base, R_BIG)],
                         out_feat.at[pl.ds(base, R_BIG)], sem_cp)
        pltpu.sync_copy(cls_hbm.at[pl.ds(base, R_BIG)],
                        cls_v.at[pl.ds(0, R_BIG)])

    @pl.when(jnp.logical_not(big))
    def _():
        pltpu.async_copy(mem_hbm.at[pl.ds(base, R_SMALL)],
                         out_feat.at[pl.ds(base, R_SMALL)], sem_cp)
        pltpu.sync_copy(cls_hbm.at[pl.ds(base, R_SMALL)],
                        cls_v.at[pl.ds(0, R_SMALL)])

    with jax.named_scope("ph_stage"):
        pltpu.sync_copy(idx_hbm, idx_v)
        pltpu.sync_copy(lab_hbm, lab_v)

    def init_body(i, _):
        wpos[pl.ds(i * L, L)] = jnp.full((L,), -1, jnp.int32)
        return 0
    with jax.named_scope("ph_init"):
        lax.fori_loop(0, WPOS_PAD // L, init_body, 0)

    def scan_body(v, _):
        x = idx_v[pl.ds(v * L, L)]
        rot[pl.ds(0, L)] = x
        rot[pl.ds(L, L)] = x
        loc = x - base
        m_in = (loc >= 0) & (loc < nrows)
        dup = jnp.zeros((L,), jnp.bool_)
        for s in range(1, L):
            xs = rot[pl.ds(s, L)]
            dup = dup | ((x == xs) & (iota < (L - s)))
        m_fin = m_in & jnp.logical_not(dup)
        posv = jnp.full((L,), v * L, jnp.int32) + iota
        plsc.store_scatter(wpos, [loc], posv, mask=m_fin)
        return 0
    with jax.named_scope("ph_scan"):
        lax.fori_loop(0, B // L, scan_body, 0)

    def comp_body(v, mt):
        wp = wpos[pl.ds(v * L, L)]
        m = wp >= jnp.zeros((L,), jnp.int32)
        cnt = jnp.sum(jnp.where(m, 1, 0).astype(jnp.int32))
        plsc.store_compressed(comp_pos.at[pl.ds(mt, L)], wp, mask=m)
        locs = jnp.full((L,), v * L, jnp.int32) + iota
        plsc.store_compressed(comp_loc.at[pl.ds(mt, L)], locs, mask=m)
        return mt + cnt
    with jax.named_scope("ph_compact"):
        M = lax.fori_loop(0, WPOS_PAD // L, comp_body, jnp.int32(0))
    Mpad = ((M + C - 1) // C) * C

    @pl.when(M > 0)
    def _():
        pv = jnp.full((L,), comp_pos[pl.ds(0, L)][0], jnp.int32)
        lv = jnp.full((L,), comp_loc[pl.ds(0, L)][0], jnp.int32)
        def pad_body(t, _):
            lanes = jnp.full((L,), t * L, jnp.int32) + iota
            mfill = lanes >= M
            plsc.store_scatter(comp_pos, [lanes], pv, mask=mfill)
            plsc.store_scatter(comp_loc, [lanes], lv, mask=mfill)
            return 0
        lax.fori_loop(M // L, Mpad // L, pad_body, 0)

    def cls_body(t, _):
        pos16 = comp_pos[pl.ds(t * L, L)]
        labs = plsc.load_gather(lab_v, [pos16])
        rows16 = comp_loc[pl.ds(t * L, L)]
        plsc.store_scatter(cls_v, [rows16], labs)
        return 0
    with jax.named_scope("ph_cls"):
        lax.fori_loop(0, Mpad // L, cls_body, 0)

    @pl.when(big)
    def _():
        pltpu.sync_copy(cls_v.at[pl.ds(0, R_BIG)],
                        out_cls.at[pl.ds(base, R_BIG)])
        pltpu.make_async_copy(mem_hbm.at[pl.ds(base, R_BIG)],
                              out_feat.at[pl.ds(base, R_BIG)], sem_cp).wait()

    @pl.when(jnp.logical_not(big))
    def _():
        pltpu.sync_copy(cls_v.at[pl.ds(0, R_SMALL)],
                        out_cls.at[pl.ds(base, R_SMALL)])
        pltpu.make_async_copy(mem_hbm.at[pl.ds(base, R_SMALL)],
                              out_feat.at[pl.ds(base, R_SMALL)], sem_cp).wait()

    def rp_body(r, _):
        g = comp_loc[pl.ds(r * L, L)] + base
        glob2d[r // 8, pl.ds((r % 8) * L, L)] = g
        return 0
    with jax.named_scope("ph_repack"):
        lax.fori_loop(0, Mpad // L, rp_body, 0)

    def ch_body(c, _):
        cpa = pltpu.async_copy(feat_hbm.at[comp_pos.at[pl.ds(c * C, C)]],
                               fbuf, sem_g0)
        cpb = pltpu.async_copy(mem_hbm.at[glob2d.at[c]], obuf, sem_g1)
        cpa.wait()
        cpb.wait()
        def fm(t, _):
            i = t // 8
            jo = (t % 8) * L
            obuf[i, pl.ds(jo, L)] = (obuf[i, pl.ds(jo, L)] * (1.0 - MOM)
                                     + fbuf[i, pl.ds(jo, L)] * MOM)
            return 0
        lax.fori_loop(0, C * (D // L), fm, 0)
        pltpu.async_copy(obuf, out_feat.at[glob2d.at[c]], sem_sc).wait()
        return 0
    with jax.named_scope("ph_chunks"):
        lax.fori_loop(0, Mpad // C, ch_body, 0)


def kernel(feature, index_target, label_target,
           target_featurememory, target_classmemory):
    k = pl.kernel(
        _body,
        out_type=(jax.ShapeDtypeStruct((T, D), jnp.float32),
                  jax.ShapeDtypeStruct((T,), jnp.int32)),
        mesh=plsc.VectorSubcoreMesh(core_axis_name="c", subcore_axis_name="s"),
        compiler_params=pltpu.CompilerParams(needs_layout_passes=False),
        scratch_types=[
            pltpu.VMEM((B,), jnp.int32),
            pltpu.VMEM((B,), jnp.int32),
            pltpu.VMEM((WPOS_PAD,), jnp.int32),
            pltpu.VMEM((COMP_PAD,), jnp.int32),
            pltpu.VMEM((COMP_PAD,), jnp.int32),
            pltpu.VMEM((COMP_PAD // C, C), jnp.int32),
            pltpu.VMEM((WPOS_PAD,), jnp.int32),
            pltpu.VMEM((C, D), jnp.float32),
            pltpu.VMEM((C, D), jnp.float32),
            pltpu.VMEM((2 * L,), jnp.int32),
            pltpu.SemaphoreType.DMA,
            pltpu.SemaphoreType.DMA,
            pltpu.SemaphoreType.DMA,
            pltpu.SemaphoreType.DMA,
        ],
    )
    return k(feature, index_target, label_target,
             target_featurememory, target_classmemory)


# speedup vs baseline: 13.6108x; 13.6108x over previous
import jax
import jax.numpy as jnp
from jax import lax
from jax.experimental import pallas as pl
from jax.experimental.pallas import tpu as pltpu
from jax.experimental.pallas import tpu_sc as plsc

B = 16384
D = 128
T = 100000
NC, NS, L = 2, 16, 16
NW = NC * NS
N_BIG = 20
R_BIG, R_SMALL = 3128, 3120
WPOS_PAD = 3136
COMP_PAD = 3200
C = 128
W = 128
NWIN = 24
REM_BIG, REM_SMALL = R_BIG - NWIN * W, R_SMALL - NWIN * W
MOM = 0.1


def _body(feat_hbm, idx_hbm, lab_hbm, mem_hbm, cls_hbm,
          out_feat, out_cls,
          idx_v, lab_v, wpos, comp_pos, comp_loc, glob2d, cls_v,
          bufa, bufb, rot, sem_ga, sem_gb, sem_sa, sem_sb):
    w = lax.axis_index("s") * NC + lax.axis_index("c")
    big = w < N_BIG
    base = jnp.where(big, w * R_BIG, N_BIG * R_BIG + (w - N_BIG) * R_SMALL)
    nrows = jnp.where(big, R_BIG, R_SMALL)
    iota = lax.iota(jnp.int32, L)

    with jax.named_scope("ph_stage"):
        pltpu.sync_copy(idx_hbm, idx_v)
        pltpu.sync_copy(lab_hbm, lab_v)

        @pl.when(big)
        def _():
            pltpu.sync_copy(cls_hbm.at[pl.ds(base, R_BIG)],
                            cls_v.at[pl.ds(0, R_BIG)])

        @pl.when(jnp.logical_not(big))
        def _():
            pltpu.sync_copy(cls_hbm.at[pl.ds(base, R_SMALL)],
                            cls_v.at[pl.ds(0, R_SMALL)])

    def init_body(i, _):
        wpos[pl.ds(i * L, L)] = jnp.full((L,), -1, jnp.int32)
        return 0
    with jax.named_scope("ph_init"):
        lax.fori_loop(0, WPOS_PAD // L, init_body, 0)

    def scan_body(v, _):
        x = idx_v[pl.ds(v * L, L)]
        rot[pl.ds(0, L)] = x
        rot[pl.ds(L, L)] = x
        loc = x - base
        m_in = (loc >= 0) & (loc < nrows)
        dup = jnp.zeros((L,), jnp.bool_)
        for s in range(1, L):
            xs = rot[pl.ds(s, L)]
            dup = dup | ((x == xs) & (iota < (L - s)))
        m_fin = m_in & jnp.logical_not(dup)
        posv = jnp.full((L,), v * L, jnp.int32) + iota
        plsc.store_scatter(wpos, [loc], posv, mask=m_fin)
        return 0

    def win_src(g):
        return mem_hbm.at[pl.ds(base + g * W, W)]

    def win_dst(g):
        return out_feat.at[pl.ds(base + g * W, W)]

    with jax.named_scope("ph_copy_scan"):
        bufs = (bufa, bufb)
        gsems = (sem_ga, sem_gb)
        ssems = (sem_sa, sem_sb)
        gd = {0: pltpu.async_copy(win_src(0), bufa, sem_ga),
              1: pltpu.async_copy(win_src(1), bufb, sem_gb)}
        nvreg = B // L
        for g in range(NWIN):
            buf, gs, ss = bufs[g % 2], gsems[g % 2], ssems[g % 2]
            lo, hi = (nvreg * g) // NWIN, (nvreg * (g + 1)) // NWIN
            lax.fori_loop(lo, hi, scan_body, 0)
            gd[g].wait()
            sd = pltpu.async_copy(buf, win_dst(g), ss)
            if g + 2 < NWIN:
                sd.wait()
                gd[g + 2] = pltpu.async_copy(win_src(g + 2), buf, gs)
            else:
                sd.wait()

        @pl.when(big)
        def _():
            pltpu.sync_copy(mem_hbm.at[pl.ds(base + NWIN * W, REM_BIG)],
                            bufa.at[pl.ds(0, REM_BIG)])
            pltpu.sync_copy(bufa.at[pl.ds(0, REM_BIG)],
                            out_feat.at[pl.ds(base + NWIN * W, REM_BIG)])

        @pl.when(jnp.logical_not(big))
        def _():
            pltpu.sync_copy(mem_hbm.at[pl.ds(base + NWIN * W, REM_SMALL)],
                            bufa.at[pl.ds(0, REM_SMALL)])
            pltpu.sync_copy(bufa.at[pl.ds(0, REM_SMALL)],
                            out_feat.at[pl.ds(base + NWIN * W, REM_SMALL)])

    def comp_body(v, mt):
        wp = wpos[pl.ds(v * L, L)]
        m = wp >= jnp.zeros((L,), jnp.int32)
        cnt = jnp.sum(jnp.where(m, 1, 0).astype(jnp.int32))
        plsc.store_compressed(comp_pos.at[pl.ds(mt, L)], wp, mask=m)
        locs = jnp.full((L,), v * L, jnp.int32) + iota
        plsc.store_compressed(comp_loc.at[pl.ds(mt, L)], locs, mask=m)
        return mt + cnt
    with jax.named_scope("ph_compact"):
        M = lax.fori_loop(0, WPOS_PAD // L, comp_body, jnp.int32(0))
    Mpad = ((M + C - 1) // C) * C

    @pl.when(M > 0)
    def _():
        pv = jnp.full((L,), comp_pos[pl.ds(0, L)][0], jnp.int32)
        lv = jnp.full((L,), comp_loc[pl.ds(0, L)][0], jnp.int32)
        def pad_body(t, _):
            lanes = jnp.full((L,), t * L, jnp.int32) + iota
            mfill = lanes >= M
            plsc.store_scatter(comp_pos, [lanes], pv, mask=mfill)
            plsc.store_scatter(comp_loc, [lanes], lv, mask=mfill)
            return 0
        lax.fori_loop(M // L, Mpad // L, pad_body, 0)

    def cls_body(t, _):
        pos16 = comp_pos[pl.ds(t * L, L)]
        labs = plsc.load_gather(lab_v, [pos16])
        rows16 = comp_loc[pl.ds(t * L, L)]
        plsc.store_scatter(cls_v, [rows16], labs)
        return 0
    with jax.named_scope("ph_cls"):
        lax.fori_loop(0, Mpad // L, cls_body, 0)

    @pl.when(big)
    def _():
        pltpu.sync_copy(cls_v.at[pl.ds(0, R_BIG)],
                        out_cls.at[pl.ds(base, R_BIG)])

    @pl.when(jnp.logical_not(big))
    def _():
        pltpu.sync_copy(cls_v.at[pl.ds(0, R_SMALL)],
                        out_cls.at[pl.ds(base, R_SMALL)])

    def rp_body(r, _):
        g = comp_loc[pl.ds(r * L, L)] + base
        glob2d[r // 8, pl.ds((r % 8) * L, L)] = g
        return 0
    with jax.named_scope("ph_repack"):
        lax.fori_loop(0, Mpad // L, rp_body, 0)

    def ch_body(c, _):
        cpa = pltpu.async_copy(feat_hbm.at[comp_pos.at[pl.ds(c * C, C)]],
                               bufa.at[pl.ds(0, C)], sem_ga)
        cpb = pltpu.async_copy(mem_hbm.at[glob2d.at[c]],
                               bufb.at[pl.ds(0, C)], sem_gb)
        cpa.wait()
        cpb.wait()
        def fm(t, _):
            i = t // 8
            jo = (t % 8) * L
            bufb[i, pl.ds(jo, L)] = (bufb[i, pl.ds(jo, L)] * (1.0 - MOM)
                                     + bufa[i, pl.ds(jo, L)] * MOM)
            return 0
        lax.fori_loop(0, C * (D // L), fm, 0)
        pltpu.async_copy(bufb.at[pl.ds(0, C)],
                         out_feat.at[glob2d.at[c]], sem_sa).wait()
        return 0
    with jax.named_scope("ph_chunks"):
        lax.fori_loop(0, Mpad // C, ch_body, 0)


def kernel(feature, index_target, label_target,
           target_featurememory, target_classmemory):
    k = pl.kernel(
        _body,
        out_type=(jax.ShapeDtypeStruct((T, D), jnp.float32),
                  jax.ShapeDtypeStruct((T,), jnp.int32)),
        mesh=plsc.VectorSubcoreMesh(core_axis_name="c", subcore_axis_name="s"),
        compiler_params=pltpu.CompilerParams(needs_layout_passes=False),
        scratch_types=[
            pltpu.VMEM((B,), jnp.int32),
            pltpu.VMEM((B,), jnp.int32),
            pltpu.VMEM((WPOS_PAD,), jnp.int32),
            pltpu.VMEM((COMP_PAD,), jnp.int32),
            pltpu.VMEM((COMP_PAD,), jnp.int32),
            pltpu.VMEM((COMP_PAD // C, C), jnp.int32),
            pltpu.VMEM((WPOS_PAD,), jnp.int32),
            pltpu.VMEM((W, D), jnp.float32),
            pltpu.VMEM((W, D), jnp.float32),
            pltpu.VMEM((2 * L,), jnp.int32),
            pltpu.SemaphoreType.DMA,
            pltpu.SemaphoreType.DMA,
            pltpu.SemaphoreType.DMA,
            pltpu.SemaphoreType.DMA,
        ],
    )
    return k(feature, index_target, label_target,
             target_featurememory, target_classmemory)


# speedup vs baseline: 16.9859x; 1.2480x over previous
import jax
import jax.numpy as jnp
from jax import lax
from jax.experimental import pallas as pl
from jax.experimental.pallas import tpu as pltpu
from jax.experimental.pallas import tpu_sc as plsc

B = 16384
D = 128
T = 100000
NC, NS, L = 2, 16, 16
NW = NC * NS
N_BIG = 20
R_BIG, R_SMALL = 3128, 3120
WPOS_PAD = 3136
COMP_PAD = 3200
C = 128
W = 128
NWIN = 24
REM_BIG, REM_SMALL = R_BIG - NWIN * W, R_SMALL - NWIN * W
MOM = 0.1


def _body(feat_hbm, idx_hbm, lab_hbm, mem_hbm, cls_hbm,
          out_feat, out_cls,
          idx_v, lab_v, wpos, comp_pos, comp_loc, glob2d, cls_v,
          bufa, bufb, rot, sem_ga, sem_gb, sem_sa, sem_sb):
    w = lax.axis_index("s") * NC + lax.axis_index("c")
    big = w < N_BIG
    base = jnp.where(big, w * R_BIG, N_BIG * R_BIG + (w - N_BIG) * R_SMALL)
    nrows = jnp.where(big, R_BIG, R_SMALL)
    iota = lax.iota(jnp.int32, L)

    with jax.named_scope("ph_stage"):
        pltpu.sync_copy(idx_hbm, idx_v)
        pltpu.sync_copy(lab_hbm, lab_v)

        @pl.when(big)
        def _():
            pltpu.sync_copy(cls_hbm.at[pl.ds(base, R_BIG)],
                            cls_v.at[pl.ds(0, R_BIG)])

        @pl.when(jnp.logical_not(big))
        def _():
            pltpu.sync_copy(cls_hbm.at[pl.ds(base, R_SMALL)],
                            cls_v.at[pl.ds(0, R_SMALL)])

    def init_body(i, _):
        wpos[pl.ds(i * L, L)] = jnp.full((L,), -1, jnp.int32)
        return 0
    with jax.named_scope("ph_init"):
        lax.fori_loop(0, WPOS_PAD // L, init_body, 0)

    def scan_body(v, _):
        x = idx_v[pl.ds(v * L, L)]
        loc = x - base
        m_in = (loc >= 0) & (loc < nrows)
        _, is_last = plsc.scan_count(x, mask=m_in)
        m_fin = m_in & is_last
        posv = jnp.full((L,), v * L, jnp.int32) + iota
        plsc.store_scatter(wpos, [loc], posv, mask=m_fin)
        return 0

    def win_src(g):
        return mem_hbm.at[pl.ds(base + g * W, W)]

    def win_dst(g):
        return out_feat.at[pl.ds(base + g * W, W)]

    with jax.named_scope("ph_copy_scan"):
        bufs = (bufa, bufb)
        gsems = (sem_ga, sem_gb)
        ssems = (sem_sa, sem_sb)
        gd = {0: pltpu.async_copy(win_src(0), bufa, sem_ga),
              1: pltpu.async_copy(win_src(1), bufb, sem_gb)}
        nvreg = B // L
        for g in range(NWIN):
            buf, gs, ss = bufs[g % 2], gsems[g % 2], ssems[g % 2]
            lo, hi = (nvreg * g) // NWIN, (nvreg * (g + 1)) // NWIN
            lax.fori_loop(lo, hi, scan_body, 0)
            gd[g].wait()
            sd = pltpu.async_copy(buf, win_dst(g), ss)
            if g + 2 < NWIN:
                sd.wait()
                gd[g + 2] = pltpu.async_copy(win_src(g + 2), buf, gs)
            else:
                sd.wait()

        @pl.when(big)
        def _():
            pltpu.sync_copy(mem_hbm.at[pl.ds(base + NWIN * W, REM_BIG)],
                            bufa.at[pl.ds(0, REM_BIG)])
            pltpu.sync_copy(bufa.at[pl.ds(0, REM_BIG)],
                            out_feat.at[pl.ds(base + NWIN * W, REM_BIG)])

        @pl.when(jnp.logical_not(big))
        def _():
            pltpu.sync_copy(mem_hbm.at[pl.ds(base + NWIN * W, REM_SMALL)],
                            bufa.at[pl.ds(0, REM_SMALL)])
            pltpu.sync_copy(bufa.at[pl.ds(0, REM_SMALL)],
                            out_feat.at[pl.ds(base + NWIN * W, REM_SMALL)])

    def comp_body(v, mt):
        wp = wpos[pl.ds(v * L, L)]
        m = wp >= jnp.zeros((L,), jnp.int32)
        cnt = jnp.sum(jnp.where(m, 1, 0).astype(jnp.int32))
        plsc.store_compressed(comp_pos.at[pl.ds(mt, L)], wp, mask=m)
        locs = jnp.full((L,), v * L, jnp.int32) + iota
        plsc.store_compressed(comp_loc.at[pl.ds(mt, L)], locs, mask=m)
        return mt + cnt
    with jax.named_scope("ph_compact"):
        M = lax.fori_loop(0, WPOS_PAD // L, comp_body, jnp.int32(0))
    Mpad = ((M + C - 1) // C) * C

    @pl.when(M > 0)
    def _():
        pv = jnp.full((L,), comp_pos[pl.ds(0, L)][0], jnp.int32)
        lv = jnp.full((L,), comp_loc[pl.ds(0, L)][0], jnp.int32)
        def pad_body(t, _):
            lanes = jnp.full((L,), t * L, jnp.int32) + iota
            mfill = lanes >= M
            plsc.store_scatter(comp_pos, [lanes], pv, mask=mfill)
            plsc.store_scatter(comp_loc, [lanes], lv, mask=mfill)
            return 0
        lax.fori_loop(M // L, Mpad // L, pad_body, 0)

    def cls_body(t, _):
        pos16 = comp_pos[pl.ds(t * L, L)]
        labs = plsc.load_gather(lab_v, [pos16])
        rows16 = comp_loc[pl.ds(t * L, L)]
        plsc.store_scatter(cls_v, [rows16], labs)
        return 0
    with jax.named_scope("ph_cls"):
        lax.fori_loop(0, Mpad // L, cls_body, 0)

    @pl.when(big)
    def _():
        pltpu.sync_copy(cls_v.at[pl.ds(0, R_BIG)],
                        out_cls.at[pl.ds(base, R_BIG)])

    @pl.when(jnp.logical_not(big))
    def _():
        pltpu.sync_copy(cls_v.at[pl.ds(0, R_SMALL)],
                        out_cls.at[pl.ds(base, R_SMALL)])

    def rp_body(r, _):
        g = comp_loc[pl.ds(r * L, L)] + base
        glob2d[r // 8, pl.ds((r % 8) * L, L)] = g
        return 0
    with jax.named_scope("ph_repack"):
        lax.fori_loop(0, Mpad // L, rp_body, 0)

    def ch_body(c, _):
        cpa = pltpu.async_copy(feat_hbm.at[comp_pos.at[pl.ds(c * C, C)]],
                               bufa.at[pl.ds(0, C)], sem_ga)
        cpb = pltpu.async_copy(mem_hbm.at[glob2d.at[c]],
                               bufb.at[pl.ds(0, C)], sem_gb)
        cpa.wait()
        cpb.wait()
        @plsc.parallel_loop(0, C * (D // L), unroll=8)
        def _(t):
            i = t // 8
            jo = (t % 8) * L
            bufb[i, pl.ds(jo, L)] = (bufb[i, pl.ds(jo, L)] * (1.0 - MOM)
                                     + bufa[i, pl.ds(jo, L)] * MOM)
        pltpu.async_copy(bufb.at[pl.ds(0, C)],
                         out_feat.at[glob2d.at[c]], sem_sa).wait()
        return 0
    with jax.named_scope("ph_chunks"):
        lax.fori_loop(0, Mpad // C, ch_body, 0)


def kernel(feature, index_target, label_target,
           target_featurememory, target_classmemory):
    k = pl.kernel(
        _body,
        out_type=(jax.ShapeDtypeStruct((T, D), jnp.float32),
                  jax.ShapeDtypeStruct((T,), jnp.int32)),
        mesh=plsc.VectorSubcoreMesh(core_axis_name="c", subcore_axis_name="s"),
        compiler_params=pltpu.CompilerParams(needs_layout_passes=False),
        scratch_types=[
            pltpu.VMEM((B,), jnp.int32),
            pltpu.VMEM((B,), jnp.int32),
            pltpu.VMEM((WPOS_PAD,), jnp.int32),
            pltpu.VMEM((COMP_PAD,), jnp.int32),
            pltpu.VMEM((COMP_PAD,), jnp.int32),
            pltpu.VMEM((COMP_PAD // C, C), jnp.int32),
            pltpu.VMEM((WPOS_PAD,), jnp.int32),
            pltpu.VMEM((W, D), jnp.float32),
            pltpu.VMEM((W, D), jnp.float32),
            pltpu.VMEM((2 * L,), jnp.int32),
            pltpu.SemaphoreType.DMA,
            pltpu.SemaphoreType.DMA,
            pltpu.SemaphoreType.DMA,
            pltpu.SemaphoreType.DMA,
        ],
    )
    return k(feature, index_target, label_target,
             target_featurememory, target_classmemory)


# speedup vs baseline: 17.6677x; 1.0401x over previous
import jax
import jax.numpy as jnp
from jax import lax
from jax.experimental import pallas as pl
from jax.experimental.pallas import tpu as pltpu
from jax.experimental.pallas import tpu_sc as plsc

B = 16384
D = 128
T = 100000
NC, NS, L = 2, 16, 16
NW = NC * NS
N_BIG = 20
R_BIG, R_SMALL = 3128, 3120
WPOS_PAD = 3136
COMP_PAD = 3200
C = 128
W = 128
NWIN = 24
REM_BIG, REM_SMALL = R_BIG - NWIN * W, R_SMALL - NWIN * W
MOM = 0.1


def _body(feat_hbm, idx_hbm, lab_hbm, mem_hbm, cls_hbm,
          out_feat, out_cls,
          idx_v, lab_v, wpos, comp_pos, comp_loc, glob2d, cls_v,
          bufa, bufb, bufc, bufd, rot,
          sem_ga, sem_gb, sem_gc, sem_gd, sem_sa, sem_sb):
    w = lax.axis_index("s") * NC + lax.axis_index("c")
    big = w < N_BIG
    base = jnp.where(big, w * R_BIG, N_BIG * R_BIG + (w - N_BIG) * R_SMALL)
    nrows = jnp.where(big, R_BIG, R_SMALL)
    iota = lax.iota(jnp.int32, L)

    with jax.named_scope("ph_stage"):
        pltpu.sync_copy(idx_hbm, idx_v)
        pltpu.sync_copy(lab_hbm, lab_v)

        @pl.when(big)
        def _():
            pltpu.sync_copy(cls_hbm.at[pl.ds(base, R_BIG)],
                            cls_v.at[pl.ds(0, R_BIG)])

        @pl.when(jnp.logical_not(big))
        def _():
            pltpu.sync_copy(cls_hbm.at[pl.ds(base, R_SMALL)],
                            cls_v.at[pl.ds(0, R_SMALL)])

    def init_body(i, _):
        wpos[pl.ds(i * L, L)] = jnp.full((L,), -1, jnp.int32)
        return 0
    with jax.named_scope("ph_init"):
        lax.fori_loop(0, WPOS_PAD // L, init_body, 0)

    def _scan_one(v):
        x = idx_v[pl.ds(v * L, L)]
        loc = x - base
        m_in = (loc >= 0) & (loc < nrows)
        _, is_last = plsc.scan_count(x, mask=m_in)
        m_fin = m_in & is_last
        posv = jnp.full((L,), v * L, jnp.int32) + iota
        return loc, posv, m_fin

    def scan_body(t, _):
        la, pa, ma = _scan_one(2 * t)
        lb, pb, mb = _scan_one(2 * t + 1)
        plsc.store_scatter(wpos, [la], pa, mask=ma)
        plsc.store_scatter(wpos, [lb], pb, mask=mb)
        return 0

    def win_src(g):
        return mem_hbm.at[pl.ds(base + g * W, W)]

    def win_dst(g):
        return out_feat.at[pl.ds(base + g * W, W)]

    with jax.named_scope("ph_copy_scan"):
        bufs = (bufa, bufb)
        gsems = (sem_ga, sem_gb)
        ssems = (sem_sa, sem_sb)
        gd = {0: pltpu.async_copy(win_src(0), bufa, sem_ga),
              1: pltpu.async_copy(win_src(1), bufb, sem_gb)}
        nvreg = B // L
        for g in range(NWIN):
            buf, gs, ss = bufs[g % 2], gsems[g % 2], ssems[g % 2]
            npair = nvreg // 2
            lo, hi = (npair * g) // NWIN, (npair * (g + 1)) // NWIN
            lax.fori_loop(lo, hi, scan_body, 0)
            gd[g].wait()
            sd = pltpu.async_copy(buf, win_dst(g), ss)
            if g + 2 < NWIN:
                sd.wait()
                gd[g + 2] = pltpu.async_copy(win_src(g + 2), buf, gs)
            else:
                sd.wait()

        @pl.when(big)
        def _():
            pltpu.sync_copy(mem_hbm.at[pl.ds(base + NWIN * W, REM_BIG)],
                            bufa.at[pl.ds(0, REM_BIG)])
            pltpu.sync_copy(bufa.at[pl.ds(0, REM_BIG)],
                            out_feat.at[pl.ds(base + NWIN * W, REM_BIG)])

        @pl.when(jnp.logical_not(big))
        def _():
            pltpu.sync_copy(mem_hbm.at[pl.ds(base + NWIN * W, REM_SMALL)],
                            bufa.at[pl.ds(0, REM_SMALL)])
            pltpu.sync_copy(bufa.at[pl.ds(0, REM_SMALL)],
                            out_feat.at[pl.ds(base + NWIN * W, REM_SMALL)])

    def comp_body(v, mt):
        wp = wpos[pl.ds(v * L, L)]
        m = wp >= jnp.zeros((L,), jnp.int32)
        cnt = jnp.sum(jnp.where(m, 1, 0).astype(jnp.int32))
        plsc.store_compressed(comp_pos.at[pl.ds(mt, L)], wp, mask=m)
        locs = jnp.full((L,), v * L, jnp.int32) + iota
        plsc.store_compressed(comp_loc.at[pl.ds(mt, L)], locs, mask=m)
        return mt + cnt
    with jax.named_scope("ph_compact"):
        M = lax.fori_loop(0, WPOS_PAD // L, comp_body, jnp.int32(0))
    Mpad = ((M + C - 1) // C) * C

    @pl.when(M > 0)
    def _():
        pv = jnp.full((L,), comp_pos[pl.ds(0, L)][0], jnp.int32)
        lv = jnp.full((L,), comp_loc[pl.ds(0, L)][0], jnp.int32)
        def pad_body(t, _):
            lanes = jnp.full((L,), t * L, jnp.int32) + iota
            mfill = lanes >= M
            plsc.store_scatter(comp_pos, [lanes], pv, mask=mfill)
            plsc.store_scatter(comp_loc, [lanes], lv, mask=mfill)
            return 0
        lax.fori_loop(M // L, Mpad // L, pad_body, 0)

    def cls_body(t, _):
        pos16 = comp_pos[pl.ds(t * L, L)]
        labs = plsc.load_gather(lab_v, [pos16])
        rows16 = comp_loc[pl.ds(t * L, L)]
        plsc.store_scatter(cls_v, [rows16], labs)
        return 0
    with jax.named_scope("ph_cls"):
        lax.fori_loop(0, Mpad // L, cls_body, 0)

    @pl.when(big)
    def _():
        pltpu.sync_copy(cls_v.at[pl.ds(0, R_BIG)],
                        out_cls.at[pl.ds(base, R_BIG)])

    @pl.when(jnp.logical_not(big))
    def _():
        pltpu.sync_copy(cls_v.at[pl.ds(0, R_SMALL)],
                        out_cls.at[pl.ds(base, R_SMALL)])

    def rp_body(r, _):
        g = comp_loc[pl.ds(r * L, L)] + base
        glob2d[r // 8, pl.ds((r % 8) * L, L)] = g
        return 0
    with jax.named_scope("ph_repack"):
        lax.fori_loop(0, Mpad // L, rp_body, 0)

    nch = Mpad // C

    def _issue_gathers(c, f, o, sf, so):
        pltpu.async_copy(feat_hbm.at[comp_pos.at[pl.ds(c * C, C)]],
                         f.at[pl.ds(0, C)], sf)
        pltpu.async_copy(mem_hbm.at[glob2d.at[c]], o.at[pl.ds(0, C)], so)

    def _process(c, f, o, sf, so, fn, on, sfn, son):
        pltpu.make_async_copy(feat_hbm.at[comp_pos.at[pl.ds(c * C, C)]],
                              f.at[pl.ds(0, C)], sf).wait()
        pltpu.make_async_copy(mem_hbm.at[glob2d.at[c]],
                              o.at[pl.ds(0, C)], so).wait()

        @pl.when(c + 1 < nch)
        def _():
            _issue_gathers(c + 1, fn, on, sfn, son)

        @plsc.parallel_loop(0, C * (D // L), unroll=8)
        def _(t):
            i = t // 8
            jo = (t % 8) * L
            o[i, pl.ds(jo, L)] = (o[i, pl.ds(jo, L)] * (1.0 - MOM)
                                  + f[i, pl.ds(jo, L)] * MOM)
        pltpu.async_copy(o.at[pl.ds(0, C)],
                         out_feat.at[glob2d.at[c]], sem_sa).wait()

    def ch_body(c, _):
        @pl.when(c % 2 == 0)
        def _():
            _process(c, bufa, bufb, sem_ga, sem_gb,
                     bufc, bufd, sem_gc, sem_gd)

        @pl.when(c % 2 == 1)
        def _():
            _process(c, bufc, bufd, sem_gc, sem_gd,
                     bufa, bufb, sem_ga, sem_gb)
        return 0

    with jax.named_scope("ph_chunks"):
        @pl.when(nch > 0)
        def _():
            _issue_gathers(0, bufa, bufb, sem_ga, sem_gb)
        lax.fori_loop(0, nch, ch_body, 0)


def kernel(feature, index_target, label_target,
           target_featurememory, target_classmemory):
    k = pl.kernel(
        _body,
        out_type=(jax.ShapeDtypeStruct((T, D), jnp.float32),
                  jax.ShapeDtypeStruct((T,), jnp.int32)),
        mesh=plsc.VectorSubcoreMesh(core_axis_name="c", subcore_axis_name="s"),
        compiler_params=pltpu.CompilerParams(needs_layout_passes=False),
        scratch_types=[
            pltpu.VMEM((B,), jnp.int32),
            pltpu.VMEM((B,), jnp.int32),
            pltpu.VMEM((WPOS_PAD,), jnp.int32),
            pltpu.VMEM((COMP_PAD,), jnp.int32),
            pltpu.VMEM((COMP_PAD,), jnp.int32),
            pltpu.VMEM((COMP_PAD // C, C), jnp.int32),
            pltpu.VMEM((WPOS_PAD,), jnp.int32),
            pltpu.VMEM((W, D), jnp.float32),
            pltpu.VMEM((W, D), jnp.float32),
            pltpu.VMEM((W, D), jnp.float32),
            pltpu.VMEM((W, D), jnp.float32),
            pltpu.VMEM((2 * L,), jnp.int32),
            pltpu.SemaphoreType.DMA,
            pltpu.SemaphoreType.DMA,
            pltpu.SemaphoreType.DMA,
            pltpu.SemaphoreType.DMA,
            pltpu.SemaphoreType.DMA,
            pltpu.SemaphoreType.DMA,
        ],
    )
    return k(feature, index_target, label_target,
             target_featurememory, target_classmemory)


# speedup vs baseline: 18.4543x; 1.0445x over previous
import jax
import jax.numpy as jnp
from jax import lax
from jax.experimental import pallas as pl
from jax.experimental.pallas import tpu as pltpu
from jax.experimental.pallas import tpu_sc as plsc

B = 16384
D = 128
T = 100000
NC, NS, L = 2, 16, 16
NW = NC * NS
N_BIG = 20
R_BIG, R_SMALL = 3128, 3120
WPOS_PAD = 3136
COMP_PAD = 3200
C = 128
W = 128
NWIN = 24
REM_BIG, REM_SMALL = R_BIG - NWIN * W, R_SMALL - NWIN * W
MOM = 0.1


def _body(feat_hbm, idx_hbm, lab_hbm, mem_hbm, cls_hbm,
          out_feat, out_cls,
          idx_v, lab_v, wpos, comp_pos, comp_loc, glob2d, cls_v,
          bufa, bufb, bufc, bufd, rot,
          sem_ga, sem_gb, sem_gc, sem_gd, sem_sa, sem_sb, sem_sc, sem_sd,
          sem_s1, sem_s2, sem_s3):
    w = lax.axis_index("s") * NC + lax.axis_index("c")
    big = w < N_BIG
    base = jnp.where(big, w * R_BIG, N_BIG * R_BIG + (w - N_BIG) * R_SMALL)
    nrows = jnp.where(big, R_BIG, R_SMALL)
    iota = lax.iota(jnp.int32, L)

    with jax.named_scope("ph_stage"):
        d_idx = pltpu.async_copy(idx_hbm, idx_v, sem_s1)
        d_lab = pltpu.async_copy(lab_hbm, lab_v, sem_s2)

        @pl.when(big)
        def _():
            pltpu.async_copy(cls_hbm.at[pl.ds(base, R_BIG)],
                             cls_v.at[pl.ds(0, R_BIG)], sem_s3)

        @pl.when(jnp.logical_not(big))
        def _():
            pltpu.async_copy(cls_hbm.at[pl.ds(base, R_SMALL)],
                             cls_v.at[pl.ds(0, R_SMALL)], sem_s3)

    def init_body(i, _):
        wpos[pl.ds(i * L, L)] = jnp.full((L,), -1, jnp.int32)
        return 0
    with jax.named_scope("ph_init"):
        lax.fori_loop(0, WPOS_PAD // L, init_body, 0)

    def _scan_one(v):
        x = idx_v[pl.ds(v * L, L)]
        loc = x - base
        m_in = (loc >= 0) & (loc < nrows)
        _, is_last = plsc.scan_count(x, mask=m_in)
        m_fin = m_in & is_last
        posv = jnp.full((L,), v * L, jnp.int32) + iota
        return loc, posv, m_fin

    def scan_body(t, _):
        la, pa, ma = _scan_one(2 * t)
        lb, pb, mb = _scan_one(2 * t + 1)
        plsc.store_scatter(wpos, [la], pa, mask=ma)
        plsc.store_scatter(wpos, [lb], pb, mask=mb)
        return 0

    def win_src(g):
        return mem_hbm.at[pl.ds(base + g * W, W)]

    def win_dst(g):
        return out_feat.at[pl.ds(base + g * W, W)]

    with jax.named_scope("ph_copy_scan"):
        bufs = (bufa, bufb, bufc, bufd)
        gsems = (sem_ga, sem_gb, sem_gc, sem_gd)
        ssems = (sem_sa, sem_sb, sem_sc, sem_sd)
        gd = {0: pltpu.async_copy(win_src(0), bufa, sem_ga),
              1: pltpu.async_copy(win_src(1), bufb, sem_gb)}
        sd = {}
        d_idx.wait()
        npair = B // L // 2
        for g in range(NWIN):
            buf = bufs[g % 4]
            lo, hi = (npair * g) // NWIN, (npair * (g + 1)) // NWIN
            lax.fori_loop(lo, hi, scan_body, 0)
            gd[g].wait()
            sd[g] = pltpu.async_copy(buf, win_dst(g), ssems[g % 4])
            if g >= 2:
                sd[g - 2].wait()
            if g + 2 < NWIN:
                gd[g + 2] = pltpu.async_copy(win_src(g + 2),
                                             bufs[(g + 2) % 4],
                                             gsems[(g + 2) % 4])
        sd[NWIN - 2].wait()
        sd[NWIN - 1].wait()

        @pl.when(big)
        def _():
            pltpu.sync_copy(mem_hbm.at[pl.ds(base + NWIN * W, REM_BIG)],
                            bufa.at[pl.ds(0, REM_BIG)])
            pltpu.sync_copy(bufa.at[pl.ds(0, REM_BIG)],
                            out_feat.at[pl.ds(base + NWIN * W, REM_BIG)])

        @pl.when(jnp.logical_not(big))
        def _():
            pltpu.sync_copy(mem_hbm.at[pl.ds(base + NWIN * W, REM_SMALL)],
                            bufa.at[pl.ds(0, REM_SMALL)])
            pltpu.sync_copy(bufa.at[pl.ds(0, REM_SMALL)],
                            out_feat.at[pl.ds(base + NWIN * W, REM_SMALL)])

    def comp_body(v, mt):
        wp = wpos[pl.ds(v * L, L)]
        m = wp >= jnp.zeros((L,), jnp.int32)
        cnt = jnp.sum(jnp.where(m, 1, 0).astype(jnp.int32))
        plsc.store_compressed(comp_pos.at[pl.ds(mt, L)], wp, mask=m)
        locs = jnp.full((L,), v * L, jnp.int32) + iota
        plsc.store_compressed(comp_loc.at[pl.ds(mt, L)], locs, mask=m)
        return mt + cnt
    with jax.named_scope("ph_compact"):
        M = lax.fori_loop(0, WPOS_PAD // L, comp_body, jnp.int32(0))
    Mpad = ((M + C - 1) // C) * C

    @pl.when(M > 0)
    def _():
        pv = jnp.full((L,), comp_pos[pl.ds(0, L)][0], jnp.int32)
        lv = jnp.full((L,), comp_loc[pl.ds(0, L)][0], jnp.int32)
        def pad_body(t, _):
            lanes = jnp.full((L,), t * L, jnp.int32) + iota
            mfill = lanes >= M
            plsc.store_scatter(comp_pos, [lanes], pv, mask=mfill)
            plsc.store_scatter(comp_loc, [lanes], lv, mask=mfill)
            return 0
        lax.fori_loop(M // L, Mpad // L, pad_body, 0)

    d_lab.wait()

    @pl.when(big)
    def _():
        pltpu.make_async_copy(cls_hbm.at[pl.ds(base, R_BIG)],
                              cls_v.at[pl.ds(0, R_BIG)], sem_s3).wait()

    @pl.when(jnp.logical_not(big))
    def _():
        pltpu.make_async_copy(cls_hbm.at[pl.ds(base, R_SMALL)],
                              cls_v.at[pl.ds(0, R_SMALL)], sem_s3).wait()

    def cls_body(t, _):
        pos16 = comp_pos[pl.ds(t * L, L)]
        labs = plsc.load_gather(lab_v, [pos16])
        rows16 = comp_loc[pl.ds(t * L, L)]
        plsc.store_scatter(cls_v, [rows16], labs)
        return 0
    with jax.named_scope("ph_cls"):
        lax.fori_loop(0, Mpad // L, cls_body, 0)

    @pl.when(big)
    def _():
        pltpu.sync_copy(cls_v.at[pl.ds(0, R_BIG)],
                        out_cls.at[pl.ds(base, R_BIG)])

    @pl.when(jnp.logical_not(big))
    def _():
        pltpu.sync_copy(cls_v.at[pl.ds(0, R_SMALL)],
                        out_cls.at[pl.ds(base, R_SMALL)])

    def rp_body(r, _):
        g = comp_loc[pl.ds(r * L, L)] + base
        glob2d[r // 8, pl.ds((r % 8) * L, L)] = g
        return 0
    with jax.named_scope("ph_repack"):
        lax.fori_loop(0, Mpad // L, rp_body, 0)

    nch = Mpad // C

    def _issue_gathers(c, f, o, sf, so):
        pltpu.async_copy(feat_hbm.at[comp_pos.at[pl.ds(c * C, C)]],
                         f.at[pl.ds(0, C)], sf)
        pltpu.async_copy(mem_hbm.at[glob2d.at[c]], o.at[pl.ds(0, C)], so)

    def _process(c, f, o, sf, so, fn, on, sfn, son, ss, on_prev, ss_prev):
        pltpu.make_async_copy(feat_hbm.at[comp_pos.at[pl.ds(c * C, C)]],
                              f.at[pl.ds(0, C)], sf).wait()
        pltpu.make_async_copy(mem_hbm.at[glob2d.at[c]],
                              o.at[pl.ds(0, C)], so).wait()

        @pl.when(c > 0)
        def _():
            pltpu.make_async_copy(on_prev.at[pl.ds(0, C)],
                                  out_feat.at[glob2d.at[c - 1]],
                                  ss_prev).wait()

        @pl.when(c + 1 < nch)
        def _():
            _issue_gathers(c + 1, fn, on, sfn, son)

        @plsc.parallel_loop(0, C * (D // L), unroll=8)
        def _(t):
            i = t // 8
            jo = (t % 8) * L
            o[i, pl.ds(jo, L)] = (o[i, pl.ds(jo, L)] * (1.0 - MOM)
                                  + f[i, pl.ds(jo, L)] * MOM)
        pltpu.async_copy(o.at[pl.ds(0, C)], out_feat.at[glob2d.at[c]], ss)

    def ch_body(c, _):
        @pl.when(c % 2 == 0)
        def _():
            _process(c, bufa, bufb, sem_ga, sem_gb,
                     bufc, bufd, sem_gc, sem_gd, sem_sa, bufd, sem_sb)

        @pl.when(c % 2 == 1)
        def _():
            _process(c, bufc, bufd, sem_gc, sem_gd,
                     bufa, bufb, sem_ga, sem_gb, sem_sb, bufb, sem_sa)
        return 0

    with jax.named_scope("ph_chunks"):
        @pl.when(nch > 0)
        def _():
            _issue_gathers(0, bufa, bufb, sem_ga, sem_gb)
        lax.fori_loop(0, nch, ch_body, 0)

        @pl.when((nch > 0) & (nch % 2 == 1))
        def _():
            pltpu.make_async_copy(bufb.at[pl.ds(0, C)],
                                  out_feat.at[glob2d.at[nch - 1]],
                                  sem_sa).wait()

        @pl.when((nch > 0) & (nch % 2 == 0))
        def _():
            pltpu.make_async_copy(bufd.at[pl.ds(0, C)],
                                  out_feat.at[glob2d.at[nch - 1]],
                                  sem_sb).wait()


def kernel(feature, index_target, label_target,
           target_featurememory, target_classmemory):
    k = pl.kernel(
        _body,
        out_type=(jax.ShapeDtypeStruct((T, D), jnp.float32),
                  jax.ShapeDtypeStruct((T,), jnp.int32)),
        mesh=plsc.VectorSubcoreMesh(core_axis_name="c", subcore_axis_name="s"),
        compiler_params=pltpu.CompilerParams(needs_layout_passes=False),
        scratch_types=[
            pltpu.VMEM((B,), jnp.int32),
            pltpu.VMEM((B,), jnp.int32),
            pltpu.VMEM((WPOS_PAD,), jnp.int32),
            pltpu.VMEM((COMP_PAD,), jnp.int32),
            pltpu.VMEM((COMP_PAD,), jnp.int32),
            pltpu.VMEM((COMP_PAD // C, C), jnp.int32),
            pltpu.VMEM((WPOS_PAD,), jnp.int32),
            pltpu.VMEM((W, D), jnp.float32),
            pltpu.VMEM((W, D), jnp.float32),
            pltpu.VMEM((W, D), jnp.float32),
            pltpu.VMEM((W, D), jnp.float32),
            pltpu.VMEM((2 * L,), jnp.int32),
            pltpu.SemaphoreType.DMA,
            pltpu.SemaphoreType.DMA,
            pltpu.SemaphoreType.DMA,
            pltpu.SemaphoreType.DMA,
            pltpu.SemaphoreType.DMA,
            pltpu.SemaphoreType.DMA,
            pltpu.SemaphoreType.DMA,
            pltpu.SemaphoreType.DMA,
            pltpu.SemaphoreType.DMA,
            pltpu.SemaphoreType.DMA,
            pltpu.SemaphoreType.DMA,
        ],
    )
    return k(feature, index_target, label_target,
             target_featurememory, target_classmemory)


# speedup vs baseline: 19.2089x; 1.0409x over previous
import jax
import jax.numpy as jnp
from jax import lax
from jax.experimental import pallas as pl
from jax.experimental.pallas import tpu as pltpu
from jax.experimental.pallas import tpu_sc as plsc

B = 16384
D = 128
T = 100000
NC, NS, L = 2, 16, 16
NW = NC * NS
N_BIG = 20
R_BIG, R_SMALL = 3128, 3120
WPOS_PAD = 3136
COMP_PAD = 3200
C = 128
W = 128
NWIN = 24
REM_BIG, REM_SMALL = R_BIG - NWIN * W, R_SMALL - NWIN * W
MOM = 0.1


def _body(feat_hbm, idx_hbm, lab_hbm, mem_hbm, cls_hbm,
          out_feat, out_cls,
          idx_v, lab_v, wpos, comp_pos, comp_loc, glob2d, cls_v,
          bufa, bufb, bufc, bufd, rot,
          sem_ga, sem_gb, sem_gc, sem_gd, sem_sa, sem_sb, sem_sc, sem_sd,
          sem_s1, sem_s2, sem_s3):
    w = lax.axis_index("s") * NC + lax.axis_index("c")
    big = w < N_BIG
    base = jnp.where(big, w * R_BIG, N_BIG * R_BIG + (w - N_BIG) * R_SMALL)
    nrows = jnp.where(big, R_BIG, R_SMALL)
    iota = lax.iota(jnp.int32, L)

    with jax.named_scope("ph_stage"):
        d_idx = pltpu.async_copy(idx_hbm, idx_v, sem_s1)
        d_lab = pltpu.async_copy(lab_hbm, lab_v, sem_s2)

        @pl.when(big)
        def _():
            pltpu.async_copy(cls_hbm.at[pl.ds(base, R_BIG)],
                             cls_v.at[pl.ds(0, R_BIG)], sem_s3)

        @pl.when(jnp.logical_not(big))
        def _():
            pltpu.async_copy(cls_hbm.at[pl.ds(base, R_SMALL)],
                             cls_v.at[pl.ds(0, R_SMALL)], sem_s3)

    def init_body(i, _):
        wpos[pl.ds(i * L, L)] = jnp.full((L,), -1, jnp.int32)
        return 0
    with jax.named_scope("ph_init"):
        lax.fori_loop(0, WPOS_PAD // L, init_body, 0)

    def _scan_one(v):
        x = idx_v[pl.ds(v * L, L)]
        loc = x - base
        m_in = (loc >= 0) & (loc < nrows)
        _, is_last = plsc.scan_count(x, mask=m_in)
        m_fin = m_in & is_last
        posv = jnp.full((L,), v * L, jnp.int32) + iota
        return loc, posv, m_fin

    def scan_body(t, _):
        la, pa, ma = _scan_one(2 * t)
        lb, pb, mb = _scan_one(2 * t + 1)
        plsc.store_scatter(wpos, [la], pa, mask=ma)
        plsc.store_scatter(wpos, [lb], pb, mask=mb)
        return 0


    def comp_body(v, mt):
        wp = wpos[pl.ds(v * L, L)]
        m = wp >= jnp.zeros((L,), jnp.int32)
        cnt = jnp.sum(jnp.where(m, 1, 0).astype(jnp.int32))
        plsc.store_compressed(comp_pos.at[pl.ds(mt, L)], wp, mask=m)
        locs = jnp.full((L,), v * L, jnp.int32) + iota
        plsc.store_compressed(comp_loc.at[pl.ds(mt, L)], locs, mask=m)
        return mt + cnt

    def cls_body(t, _):
        pos16 = comp_pos[pl.ds(t * L, L)]
        labs = plsc.load_gather(lab_v, [pos16])
        rows16 = comp_loc[pl.ds(t * L, L)]
        plsc.store_scatter(cls_v, [rows16], labs)
        return 0

    def rp_body(r, _):
        g = comp_loc[pl.ds(r * L, L)] + base
        glob2d[r // 8, pl.ds((r % 8) * L, L)] = g
        return 0

    def win_src(g):
        return mem_hbm.at[pl.ds(base + g * W, W)]

    def win_dst(g):
        return out_feat.at[pl.ds(base + g * W, W)]

    with jax.named_scope("ph_copy_scan"):
        bufs = (bufa, bufb, bufc, bufd)
        gsems = (sem_ga, sem_gb, sem_gc, sem_gd)
        ssems = (sem_sa, sem_sb, sem_sc, sem_sd)
        gd = {0: pltpu.async_copy(win_src(0), bufa, sem_ga),
              1: pltpu.async_copy(win_src(1), bufb, sem_gb)}
        sd = {}
        d_idx.wait()
        npair = B // L // 2
        NSC, NCO = 12, 6
        M = jnp.int32(0)
        Mpad = jnp.int32(0)
        for g in range(NWIN):
            buf = bufs[g % 4]
            if g < NSC:
                lo, hi = (npair * g) // NSC, (npair * (g + 1)) // NSC
                lax.fori_loop(lo, hi, scan_body, 0)
            elif g < NSC + NCO:
                k = g - NSC
                nc = WPOS_PAD // L
                lo, hi = (nc * k) // NCO, (nc * (k + 1)) // NCO
                M = lax.fori_loop(lo, hi, comp_body, M)
                if g == NSC + NCO - 1:
                    Mpad = ((M + C - 1) // C) * C
            elif g == 18:
                @pl.when(M > 0)
                def _():
                    pv = jnp.full((L,), comp_pos[pl.ds(0, L)][0], jnp.int32)
                    lv = jnp.full((L,), comp_loc[pl.ds(0, L)][0], jnp.int32)
                    def pad_body(t, _):
                        lanes = jnp.full((L,), t * L, jnp.int32) + iota
                        mfill = lanes >= M
                        plsc.store_scatter(comp_pos, [lanes], pv, mask=mfill)
                        plsc.store_scatter(comp_loc, [lanes], lv, mask=mfill)
                        return 0
                    lax.fori_loop(M // L, Mpad // L, pad_body, 0)
            elif g in (19, 20, 21):
                if g == 19:
                    d_lab.wait()

                    @pl.when(big)
                    def _():
                        pltpu.make_async_copy(
                            cls_hbm.at[pl.ds(base, R_BIG)],
                            cls_v.at[pl.ds(0, R_BIG)], sem_s3).wait()

                    @pl.when(jnp.logical_not(big))
                    def _():
                        pltpu.make_async_copy(
                            cls_hbm.at[pl.ds(base, R_SMALL)],
                            cls_v.at[pl.ds(0, R_SMALL)], sem_s3).wait()
                k = g - 19
                ncv = Mpad // L
                lax.fori_loop((ncv * k) // 3, (ncv * (k + 1)) // 3,
                              cls_body, 0)
            elif g == 22:
                @pl.when(big)
                def _():
                    pltpu.sync_copy(cls_v.at[pl.ds(0, R_BIG)],
                                    out_cls.at[pl.ds(base, R_BIG)])

                @pl.when(jnp.logical_not(big))
                def _():
                    pltpu.sync_copy(cls_v.at[pl.ds(0, R_SMALL)],
                                    out_cls.at[pl.ds(base, R_SMALL)])
            else:
                lax.fori_loop(0, Mpad // L, rp_body, 0)
            gd[g].wait()
            sd[g] = pltpu.async_copy(buf, win_dst(g), ssems[g % 4])
            if g >= 2:
                sd[g - 2].wait()
            if g + 2 < NWIN:
                gd[g + 2] = pltpu.async_copy(win_src(g + 2),
                                             bufs[(g + 2) % 4],
                                             gsems[(g + 2) % 4])
        sd[NWIN - 2].wait()
        sd[NWIN - 1].wait()

        @pl.when(big)
        def _():
            pltpu.sync_copy(mem_hbm.at[pl.ds(base + NWIN * W, REM_BIG)],
                            bufa.at[pl.ds(0, REM_BIG)])
            pltpu.sync_copy(bufa.at[pl.ds(0, REM_BIG)],
                            out_feat.at[pl.ds(base + NWIN * W, REM_BIG)])

        @pl.when(jnp.logical_not(big))
        def _():
            pltpu.sync_copy(mem_hbm.at[pl.ds(base + NWIN * W, REM_SMALL)],
                            bufa.at[pl.ds(0, REM_SMALL)])
            pltpu.sync_copy(bufa.at[pl.ds(0, REM_SMALL)],
                            out_feat.at[pl.ds(base + NWIN * W, REM_SMALL)])

    nch = Mpad // C

    def _issue_gathers(c, f, o, sf, so):
        pltpu.async_copy(feat_hbm.at[comp_pos.at[pl.ds(c * C, C)]],
                         f.at[pl.ds(0, C)], sf)
        pltpu.async_copy(mem_hbm.at[glob2d.at[c]], o.at[pl.ds(0, C)], so)

    def _process(c, f, o, sf, so, fn, on, sfn, son, ss, on_prev, ss_prev):
        pltpu.make_async_copy(feat_hbm.at[comp_pos.at[pl.ds(c * C, C)]],
                              f.at[pl.ds(0, C)], sf).wait()
        pltpu.make_async_copy(mem_hbm.at[glob2d.at[c]],
                              o.at[pl.ds(0, C)], so).wait()

        @pl.when(c > 0)
        def _():
            pltpu.make_async_copy(on_prev.at[pl.ds(0, C)],
                                  out_feat.at[glob2d.at[c - 1]],
                                  ss_prev).wait()

        @pl.when(c + 1 < nch)
        def _():
            _issue_gathers(c + 1, fn, on, sfn, son)

        @plsc.parallel_loop(0, C * (D // L), unroll=8)
        def _(t):
            i = t // 8
            jo = (t % 8) * L
            o[i, pl.ds(jo, L)] = (o[i, pl.ds(jo, L)] * (1.0 - MOM)
                                  + f[i, pl.ds(jo, L)] * MOM)
        pltpu.async_copy(o.at[pl.ds(0, C)], out_feat.at[glob2d.at[c]], ss)

    def ch_body(c, _):
        @pl.when(c % 2 == 0)
        def _():
            _process(c, bufa, bufb, sem_ga, sem_gb,
                     bufc, bufd, sem_gc, sem_gd, sem_sa, bufd, sem_sb)

        @pl.when(c % 2 == 1)
        def _():
            _process(c, bufc, bufd, sem_gc, sem_gd,
                     bufa, bufb, sem_ga, sem_gb, sem_sb, bufb, sem_sa)
        return 0

    with jax.named_scope("ph_chunks"):
        @pl.when(nch > 0)
        def _():
            _issue_gathers(0, bufa, bufb, sem_ga, sem_gb)
        lax.fori_loop(0, nch, ch_body, 0)

        @pl.when((nch > 0) & (nch % 2 == 1))
        def _():
            pltpu.make_async_copy(bufb.at[pl.ds(0, C)],
                                  out_feat.at[glob2d.at[nch - 1]],
                                  sem_sa).wait()

        @pl.when((nch > 0) & (nch % 2 == 0))
        def _():
            pltpu.make_async_copy(bufd.at[pl.ds(0, C)],
                                  out_feat.at[glob2d.at[nch - 1]],
                                  sem_sb).wait()


def kernel(feature, index_target, label_target,
           target_featurememory, target_classmemory):
    k = pl.kernel(
        _body,
        out_type=(jax.ShapeDtypeStruct((T, D), jnp.float32),
                  jax.ShapeDtypeStruct((T,), jnp.int32)),
        mesh=plsc.VectorSubcoreMesh(core_axis_name="c", subcore_axis_name="s"),
        compiler_params=pltpu.CompilerParams(needs_layout_passes=False),
        scratch_types=[
            pltpu.VMEM((B,), jnp.int32),
            pltpu.VMEM((B,), jnp.int32),
            pltpu.VMEM((WPOS_PAD,), jnp.int32),
            pltpu.VMEM((COMP_PAD,), jnp.int32),
            pltpu.VMEM((COMP_PAD,), jnp.int32),
            pltpu.VMEM((COMP_PAD // C, C), jnp.int32),
            pltpu.VMEM((WPOS_PAD,), jnp.int32),
            pltpu.VMEM((W, D), jnp.float32),
            pltpu.VMEM((W, D), jnp.float32),
            pltpu.VMEM((W, D), jnp.float32),
            pltpu.VMEM((W, D), jnp.float32),
            pltpu.VMEM((2 * L,), jnp.int32),
            pltpu.SemaphoreType.DMA,
            pltpu.SemaphoreType.DMA,
            pltpu.SemaphoreType.DMA,
            pltpu.SemaphoreType.DMA,
            pltpu.SemaphoreType.DMA,
            pltpu.SemaphoreType.DMA,
            pltpu.SemaphoreType.DMA,
            pltpu.SemaphoreType.DMA,
            pltpu.SemaphoreType.DMA,
            pltpu.SemaphoreType.DMA,
            pltpu.SemaphoreType.DMA,
        ],
    )
    return k(feature, index_target, label_target,
             target_featurememory, target_classmemory)
